# baseline (speedup 1.0000x reference)
import functools

import jax
import jax.numpy as jnp
from jax import lax
from jax.experimental import pallas as pl
from jax.experimental.pallas import tpu as pltpu
from jax.experimental.pallas import tpu_sc as plsc

T = 4096
H = 1024
E = 8
FF = 2816
KTOP = 2
AUX_COEF = 0.01
Z_COEF = 0.001

BLK = 256
NB = (KTOP * T) // BLK + E - 1
NPAD = NB * BLK
NFF = 2
FFC = FF // NFF
TB = 512
NTB = T // TB
WV = 128

_f32 = jnp.float32
_i32 = jnp.int32


def _router_body(x_ref, rw_ref, d_ref, w_ref, be_ref, lb_ref, z_ref,
                 oh1_ref, oh2_ref):
    x = x_ref[...]
    rw = rw_ref[...]
    logits = lax.dot_general(x, rw, (((1,), (1,)), ((), ())),
                             preferred_element_type=_f32)
    ii = lax.broadcasted_iota(_i32, (T, E), 1)
    m1 = jnp.max(logits, axis=1, keepdims=True)
    i1 = jnp.min(jnp.where(logits == m1, ii, E), axis=1, keepdims=True)
    oh1 = (ii == i1).astype(_f32)
    masked = jnp.where(oh1 > 0.5, -jnp.inf, logits)
    m2 = jnp.max(masked, axis=1, keepdims=True)
    i2 = jnp.min(jnp.where(masked == m2, ii, E), axis=1, keepdims=True)
    oh2 = (ii == i2).astype(_f32)
    oh1_ref[...] = oh1
    oh2_ref[...] = oh2

    ew = jnp.exp(m2 - m1)
    w1 = 1.0 / (1.0 + ew)
    w2 = ew / (1.0 + ew)
    w_ref[0:1, :] = w1.reshape(1, T)
    w_ref[1:2, :] = w2.reshape(1, T)

    se = jnp.sum(jnp.exp(logits - m1), axis=1, keepdims=True)
    lse = m1 + jnp.log(se)
    z_ref[...] = jnp.reshape(Z_COEF * jnp.sum(lse * lse) / T, (1, 1))

    counts = jnp.sum(oh1 + oh2, axis=0, keepdims=True)
    usage = counts / T
    mean_u = jnp.sum(usage) / E
    var_u = jnp.sum((usage - mean_u) ** 2) / (E - 1)
    lb_ref[...] = jnp.reshape(AUX_COEF * E * var_u, (1, 1))

    nb_e = jnp.floor((counts + (BLK - 1)) / BLK)
    er = lax.broadcasted_iota(_i32, (E, E), 0)
    ec = lax.broadcasted_iota(_i32, (E, E), 1)
    mlt = (er < ec).astype(_f32)
    cumx = lax.dot_general(nb_e, mlt, (((1,), (0,)), ((), ())),
                           preferred_element_type=_f32,
                           precision=lax.Precision.HIGHEST)
    pad_off = BLK * cumx
    c0tot = jnp.sum(oh1, axis=0, keepdims=True)

    cum_incl = jnp.round(cumx + nb_e).astype(_i32).reshape(E, 1)
    bidx = lax.broadcasted_iota(_i32, (E, 64), 1)
    be = jnp.sum((bidx >= cum_incl).astype(_i32), axis=0, keepdims=True)
    be_ref[0:1, :] = jnp.minimum(be, E - 1)

    tr = lax.broadcasted_iota(_i32, (TB, TB), 0)
    tc = lax.broadcasted_iota(_i32, (TB, TB), 1)
    tril = (tr > tc).astype(_f32)

    def body(b, carry):
        run0, run1 = carry
        sl = pl.ds(b * TB, TB)
        o1 = oh1_ref[sl, :]
        o2 = oh2_ref[sl, :]
        r0 = lax.dot_general(tril, o1, (((1,), (0,)), ((), ())),
                             preferred_element_type=_f32,
                             precision=lax.Precision.HIGHEST)
        r1 = lax.dot_general(tril, o2, (((1,), (0,)), ((), ())),
                             preferred_element_type=_f32,
                             precision=lax.Precision.HIGHEST)
        d0 = jnp.sum(o1 * (pad_off + run0) + o1 * r0, axis=1)
        d1 = jnp.sum(o2 * (pad_off + c0tot + run1) + o2 * r1, axis=1)
        d_ref[0:1, sl] = jnp.round(d0).astype(_i32).reshape(1, TB)
        d_ref[1:2, sl] = jnp.round(d1).astype(_i32).reshape(1, TB)
        return (run0 + jnp.sum(o1, axis=0, keepdims=True),
                run1 + jnp.sum(o2, axis=0, keepdims=True))

    lax.fori_loop(0, NTB, body, (jnp.zeros((1, E), _f32),
                                 jnp.zeros((1, E), _f32)))


def _router(x, router_w):
    return pl.pallas_call(
        _router_body,
        out_shape=(
            jax.ShapeDtypeStruct((8, T), _i32),
            jax.ShapeDtypeStruct((8, T), _f32),
            jax.ShapeDtypeStruct((8, 64), _i32),
            jax.ShapeDtypeStruct((1, 1), _f32),
            jax.ShapeDtypeStruct((1, 1), _f32),
        ),
        scratch_shapes=[pltpu.VMEM((T, E), _f32), pltpu.VMEM((T, E), _f32)],
    )(x, router_w)


_PPW = (KTOP * T) // 32
_CH = 32
_NCH = _PPW // _CH

def _dispatch_body(x_hbm, w_hbm, d_hbm, xs_hbm, ws_hbm,
                   idx0, idx1, rows0, rows1, w0, w1,
                   semi0, semi1, semo0, semo1):
    wid = lax.axis_index("s") * 2 + lax.axis_index("c")
    base = wid * _PPW
    idx = (idx0, idx1)
    rows = (rows0, rows1)
    wv = (w0, w1)
    semi = (semi0, semi1)
    semo = (semo0, semo1)

    def start_in(c, b):
        off = base + c * _CH
        src = lax.rem(off, T)
        return (pltpu.async_copy(d_hbm.at[pl.ds(off, _CH)], idx[b], semi[b]),
                pltpu.async_copy(x_hbm.at[pl.ds(src, _CH)], rows[b], semi[b]),
                pltpu.async_copy(w_hbm.at[pl.ds(off, _CH)], wv[b], semi[b]))

    pend_in = {0: start_in(0, 0)}
    pend_out = {}
    for c in range(_NCH):
        b = c & 1
        for cp in pend_in.pop(c):
            cp.wait()
        if c + 1 < _NCH:
            if c - 1 >= 0:
                for cp in pend_out.pop(c - 1):
                    cp.wait()
            pend_in[c + 1] = start_in(c + 1, (c + 1) & 1)
        pend_out[c] = (
            pltpu.async_copy(rows[b], xs_hbm.at[idx[b]], semo[b]),
            pltpu.async_copy(wv[b], ws_hbm.at[idx[b]], semo[b]))
    for k in sorted(pend_out):
        for cp in pend_out[k]:
            cp.wait()


def _dispatch(x, w_pairs, d_pairs):
    mesh = plsc.VectorSubcoreMesh(core_axis_name="c", subcore_axis_name="s")
    return pl.kernel(
        _dispatch_body,
        out_type=(jax.ShapeDtypeStruct((NPAD, H), _f32),
                  jax.ShapeDtypeStruct((NPAD, WV), _f32)),
        mesh=mesh,
        scratch_types=[pltpu.VMEM((_CH,), _i32),
                       pltpu.VMEM((_CH,), _i32),
                       pltpu.VMEM((_CH, H), _f32),
                       pltpu.VMEM((_CH, H), _f32),
                       pltpu.VMEM((_CH, WV), _f32),
                       pltpu.VMEM((_CH, WV), _f32),
                       pltpu.SemaphoreType.DMA,
                       pltpu.SemaphoreType.DMA,
                       pltpu.SemaphoreType.DMA,
                       pltpu.SemaphoreType.DMA],
    )(x, w_pairs, d_pairs)


def _cast_mid_body(i_ref, o_ref):
    o_ref[...] = i_ref[...].astype(jnp.bfloat16)


def _cast_bf16(arr, mid_axis):
    e, d1, d2 = arr.shape
    nc = 11
    if mid_axis == 1:
        blk, imap = (1, d1 // nc, d2), (lambda i, j: (i, j, 0))
    else:
        blk, imap = (1, d1, d2 // nc), (lambda i, j: (i, 0, j))
    return pl.pallas_call(
        _cast_mid_body,
        grid=(e, nc),
        in_specs=[pl.BlockSpec(blk, imap)],
        out_specs=pl.BlockSpec(blk, imap),
        out_shape=jax.ShapeDtypeStruct(arr.shape, jnp.bfloat16),
    )(arr)


def _ffn_body(be_ref, x_ref, gw_ref, uw_ref, dw_ref, w_ref, y_ref):
    x = x_ref[...].astype(jnp.bfloat16)
    g = lax.dot_general(x, gw_ref[0], (((1,), (1,)), ((), ())),
                        preferred_element_type=_f32)
    u = lax.dot_general(x, uw_ref[0], (((1,), (1,)), ((), ())),
                        preferred_element_type=_f32)
    act = (g * (1.0 / (1.0 + jnp.exp(-g))) * u).astype(jnp.bfloat16)
    y = lax.dot_general(act, dw_ref[0], (((1,), (1,)), ((), ())),
                        preferred_element_type=_f32)
    y_ref[...] = y * w_ref[:, 0:1]


def _ffn(be, xs, ws, gate_w, up_w, down_w):
    return pl.pallas_call(
        _ffn_body,
        grid_spec=pltpu.PrefetchScalarGridSpec(
            num_scalar_prefetch=1,
            grid=(NB,),
            in_specs=[
                pl.BlockSpec((BLK, H), lambda i, be: (i, 0)),
                pl.BlockSpec((1, FF, H), lambda i, be: (be[i], 0, 0)),
                pl.BlockSpec((1, FF, H), lambda i, be: (be[i], 0, 0)),
                pl.BlockSpec((1, H, FF), lambda i, be: (be[i], 0, 0)),
                pl.BlockSpec((BLK, WV), lambda i, be: (i, 0)),
            ],
            out_specs=pl.BlockSpec((BLK, H), lambda i, be: (i, 0)),
        ),
        out_shape=jax.ShapeDtypeStruct((NPAD, H), _f32),
    )(be, xs, gate_w, up_w, down_w, ws)


_TPW = T // 32
_CT = 16
_NCT = _TPW // _CT

def _combine_body(y_hbm, d_hbm, out_hbm,
                  i0a, i0b, i1a, i1b, r0a, r0b, r1a, r1b,
                  semi0, semi1, semo0, semo1):
    wid = lax.axis_index("s") * 2 + lax.axis_index("c")
    tb = wid * _TPW
    idx0 = (i0a, i0b)
    idx1 = (i1a, i1b)
    r0 = (r0a, r0b)
    r1 = (r1a, r1b)
    semi = (semi0, semi1)
    semo = (semo0, semo1)

    def start_in(c, b):
        t0 = tb + c * _CT
        cpa = pltpu.async_copy(d_hbm.at[pl.ds(t0, _CT)], idx0[b], semi[b])
        cpb = pltpu.async_copy(d_hbm.at[pl.ds(T + t0, _CT)], idx1[b], semi[b])
        cpa.wait()
        cpb.wait()
        return (pltpu.async_copy(y_hbm.at[idx0[b]], r0[b], semi[b]),
                pltpu.async_copy(y_hbm.at[idx1[b]], r1[b], semi[b]))

    pend_in = {0: start_in(0, 0)}
    pend_out = {}
    for c in range(_NCT):
        b = c & 1
        for cp in pend_in.pop(c):
            cp.wait()
        if c + 1 < _NCT:
            if c - 1 >= 0:
                pend_out.pop(c - 1).wait()
            pend_in[c + 1] = start_in(c + 1, (c + 1) & 1)

        def row_body(r, carry):
            for k in range(H // 16):
                sl = pl.ds(k * 16, 16)
                r0[b][r, sl] = r0[b][r, sl] + r1[b][r, sl]
            return carry

        lax.fori_loop(0, _CT, row_body, 0)
        pend_out[c] = pltpu.async_copy(
            r0[b], out_hbm.at[pl.ds(tb + c * _CT, _CT)], semo[b])
    for k in sorted(pend_out):
        pend_out[k].wait()


def _combine(y, d_pairs):
    mesh = plsc.VectorSubcoreMesh(core_axis_name="c", subcore_axis_name="s")
    return pl.kernel(
        _combine_body,
        out_type=jax.ShapeDtypeStruct((T, H), _f32),
        mesh=mesh,
        scratch_types=[pltpu.VMEM((_CT,), _i32),
                       pltpu.VMEM((_CT,), _i32),
                       pltpu.VMEM((_CT,), _i32),
                       pltpu.VMEM((_CT,), _i32),
                       pltpu.VMEM((_CT, H), _f32),
                       pltpu.VMEM((_CT, H), _f32),
                       pltpu.VMEM((_CT, H), _f32),
                       pltpu.VMEM((_CT, H), _f32),
                       pltpu.SemaphoreType.DMA,
                       pltpu.SemaphoreType.DMA,
                       pltpu.SemaphoreType.DMA,
                       pltpu.SemaphoreType.DMA],
    )(y, d_pairs)


def kernel(hidden_states, router_w, gate_w, up_w, down_w):
    b, s, h = hidden_states.shape
    x = hidden_states.reshape(-1, h)
    d, w, be, lb, z = _router(x, router_w)
    d_pairs = jnp.concatenate([d[0], d[1]])
    w_pairs = jnp.broadcast_to(
        jnp.concatenate([w[0], w[1]])[:, None], (KTOP * T, WV))
    be_vec = be[0, :NB]
    xs, ws = _dispatch(x, w_pairs, d_pairs)
    y = _ffn(be_vec, xs, ws, _cast_bf16(gate_w, 1), _cast_bf16(up_w, 1),
             _cast_bf16(down_w, 2))
    final = _combine(y, d_pairs)
    return (final.reshape(b, s, h), lb.reshape(()), z.reshape(()))


# speedup vs baseline: 1.1905x; 1.1905x over previous
import functools

import jax
import jax.numpy as jnp
from jax import lax
from jax.experimental import pallas as pl
from jax.experimental.pallas import tpu as pltpu
from jax.experimental.pallas import tpu_sc as plsc

T = 4096
H = 1024
E = 8
FF = 2816
KTOP = 2
AUX_COEF = 0.01
Z_COEF = 0.001

BLK = 256
NB = (KTOP * T) // BLK + E - 1
NPAD = NB * BLK
NFF = 2
FFC = FF // NFF
TB = 512
NTB = T // TB
WV = 128

_f32 = jnp.float32
_i32 = jnp.int32


def _router_body(x_ref, rw_ref, d_ref, w_ref, be_ref, lb_ref, z_ref,
                 oh1_ref, oh2_ref):
    x = x_ref[...]
    rw = rw_ref[...]
    logits = lax.dot_general(x, rw, (((1,), (1,)), ((), ())),
                             preferred_element_type=_f32)
    ii = lax.broadcasted_iota(_i32, (T, E), 1)
    m1 = jnp.max(logits, axis=1, keepdims=True)
    i1 = jnp.min(jnp.where(logits == m1, ii, E), axis=1, keepdims=True)
    oh1 = (ii == i1).astype(_f32)
    masked = jnp.where(oh1 > 0.5, -jnp.inf, logits)
    m2 = jnp.max(masked, axis=1, keepdims=True)
    i2 = jnp.min(jnp.where(masked == m2, ii, E), axis=1, keepdims=True)
    oh2 = (ii == i2).astype(_f32)
    oh1_ref[...] = oh1
    oh2_ref[...] = oh2

    ew = jnp.exp(m2 - m1)
    w1 = 1.0 / (1.0 + ew)
    w2 = ew / (1.0 + ew)
    w_ref[0:1, :] = w1.reshape(1, T)
    w_ref[1:2, :] = w2.reshape(1, T)

    se = jnp.sum(jnp.exp(logits - m1), axis=1, keepdims=True)
    lse = m1 + jnp.log(se)
    z_ref[...] = jnp.reshape(Z_COEF * jnp.sum(lse * lse) / T, (1, 1))

    counts = jnp.sum(oh1 + oh2, axis=0, keepdims=True)
    usage = counts / T
    mean_u = jnp.sum(usage) / E
    var_u = jnp.sum((usage - mean_u) ** 2) / (E - 1)
    lb_ref[...] = jnp.reshape(AUX_COEF * E * var_u, (1, 1))

    nb_e = jnp.floor((counts + (BLK - 1)) / BLK)
    er = lax.broadcasted_iota(_i32, (E, E), 0)
    ec = lax.broadcasted_iota(_i32, (E, E), 1)
    mlt = (er < ec).astype(_f32)
    cumx = lax.dot_general(nb_e, mlt, (((1,), (0,)), ((), ())),
                           preferred_element_type=_f32,
                           precision=lax.Precision.HIGHEST)
    pad_off = BLK * cumx
    c0tot = jnp.sum(oh1, axis=0, keepdims=True)

    cum_incl = jnp.round(cumx + nb_e).astype(_i32).reshape(E, 1)
    bidx = lax.broadcasted_iota(_i32, (E, 64), 1)
    be = jnp.sum((bidx >= cum_incl).astype(_i32), axis=0, keepdims=True)
    be_ref[0:1, :] = jnp.minimum(be, E - 1)

    tr = lax.broadcasted_iota(_i32, (TB, TB), 0)
    tc = lax.broadcasted_iota(_i32, (TB, TB), 1)
    tril = (tr > tc).astype(_f32)

    def body(b, carry):
        run0, run1 = carry
        sl = pl.ds(b * TB, TB)
        o1 = oh1_ref[sl, :]
        o2 = oh2_ref[sl, :]
        r0 = lax.dot_general(tril, o1, (((1,), (0,)), ((), ())),
                             preferred_element_type=_f32,
                             precision=lax.Precision.HIGHEST)
        r1 = lax.dot_general(tril, o2, (((1,), (0,)), ((), ())),
                             preferred_element_type=_f32,
                             precision=lax.Precision.HIGHEST)
        d0 = jnp.sum(o1 * (pad_off + run0) + o1 * r0, axis=1)
        d1 = jnp.sum(o2 * (pad_off + c0tot + run1) + o2 * r1, axis=1)
        d_ref[0:1, sl] = jnp.round(d0).astype(_i32).reshape(1, TB)
        d_ref[1:2, sl] = jnp.round(d1).astype(_i32).reshape(1, TB)
        return (run0 + jnp.sum(o1, axis=0, keepdims=True),
                run1 + jnp.sum(o2, axis=0, keepdims=True))

    lax.fori_loop(0, NTB, body, (jnp.zeros((1, E), _f32),
                                 jnp.zeros((1, E), _f32)))


def _router(x, router_w):
    return pl.pallas_call(
        _router_body,
        out_shape=(
            jax.ShapeDtypeStruct((8, T), _i32),
            jax.ShapeDtypeStruct((8, T), _f32),
            jax.ShapeDtypeStruct((8, 64), _i32),
            jax.ShapeDtypeStruct((1, 1), _f32),
            jax.ShapeDtypeStruct((1, 1), _f32),
        ),
        scratch_shapes=[pltpu.VMEM((T, E), _f32), pltpu.VMEM((T, E), _f32)],
    )(x, router_w)


_PPW = (KTOP * T) // 32
_CH = 32
_NCH = _PPW // _CH

def _dispatch_body(x_hbm, w_hbm, d_hbm, xs_hbm, ws_hbm,
                   idx0, idx1, rows0, rows1, w0, w1,
                   semi0, semi1, semo0, semo1):
    wid = lax.axis_index("s") * 2 + lax.axis_index("c")
    base = wid * _PPW
    idx = (idx0, idx1)
    rows = (rows0, rows1)
    wv = (w0, w1)
    semi = (semi0, semi1)
    semo = (semo0, semo1)

    def start_in(c, b):
        off = base + c * _CH
        src = lax.rem(off, T)
        return (pltpu.async_copy(d_hbm.at[pl.ds(off, _CH)], idx[b], semi[b]),
                pltpu.async_copy(x_hbm.at[pl.ds(src, _CH)], rows[b], semi[b]),
                pltpu.async_copy(w_hbm.at[pl.ds(off, _CH)], wv[b], semi[b]))

    pend_in = {0: start_in(0, 0)}
    pend_out = {}
    for c in range(_NCH):
        b = c & 1
        for cp in pend_in.pop(c):
            cp.wait()
        if c + 1 < _NCH:
            if c - 1 >= 0:
                for cp in pend_out.pop(c - 1):
                    cp.wait()
            pend_in[c + 1] = start_in(c + 1, (c + 1) & 1)
        pend_out[c] = (
            pltpu.async_copy(rows[b], xs_hbm.at[idx[b]], semo[b]),
            pltpu.async_copy(wv[b], ws_hbm.at[idx[b]], semo[b]))
    for k in sorted(pend_out):
        for cp in pend_out[k]:
            cp.wait()


def _dispatch(x, w_pairs, d_pairs):
    mesh = plsc.VectorSubcoreMesh(core_axis_name="c", subcore_axis_name="s")
    return pl.kernel(
        _dispatch_body,
        out_type=(jax.ShapeDtypeStruct((NPAD, H), _f32),
                  jax.ShapeDtypeStruct((NPAD, WV), _f32)),
        mesh=mesh,
        scratch_types=[pltpu.VMEM((_CH,), _i32),
                       pltpu.VMEM((_CH,), _i32),
                       pltpu.VMEM((_CH, H), _f32),
                       pltpu.VMEM((_CH, H), _f32),
                       pltpu.VMEM((_CH, WV), _f32),
                       pltpu.VMEM((_CH, WV), _f32),
                       pltpu.SemaphoreType.DMA,
                       pltpu.SemaphoreType.DMA,
                       pltpu.SemaphoreType.DMA,
                       pltpu.SemaphoreType.DMA],
    )(x, w_pairs, d_pairs)


def _cast_mid_body(i_ref, o_ref):
    o_ref[...] = i_ref[...].astype(jnp.bfloat16)


def _cast_bf16(arr, mid_axis):
    e, d1, d2 = arr.shape
    nc = NFF
    if mid_axis == 1:
        blk, imap = (1, d1 // nc, d2), (lambda i, j: (i, j, 0))
    else:
        blk, imap = (1, d1, d2 // nc), (lambda i, j: (i, 0, j))
    return pl.pallas_call(
        _cast_mid_body,
        grid=(e, nc),
        in_specs=[pl.BlockSpec(blk, imap)],
        out_specs=pl.BlockSpec(blk, imap),
        out_shape=jax.ShapeDtypeStruct(arr.shape, jnp.bfloat16),
    )(arr)


def _ffn_body(be_ref, x_ref, gw_ref, uw_ref, dw_ref, w_ref, y_ref):
    x = x_ref[...].astype(jnp.bfloat16)
    g = lax.dot_general(x, gw_ref[0], (((1,), (1,)), ((), ())),
                        preferred_element_type=_f32)
    u = lax.dot_general(x, uw_ref[0], (((1,), (1,)), ((), ())),
                        preferred_element_type=_f32)
    act = (g * (1.0 / (1.0 + jnp.exp(-g))) * u).astype(jnp.bfloat16)
    y = lax.dot_general(act, dw_ref[0], (((1,), (1,)), ((), ())),
                        preferred_element_type=_f32)
    y_ref[...] = y * w_ref[:, 0:1]


def _ffn(be, xs, ws, gate_w, up_w, down_w):
    return pl.pallas_call(
        _ffn_body,
        grid_spec=pltpu.PrefetchScalarGridSpec(
            num_scalar_prefetch=1,
            grid=(NB,),
            in_specs=[
                pl.BlockSpec((BLK, H), lambda i, be: (i, 0)),
                pl.BlockSpec((1, FF, H), lambda i, be: (be[i], 0, 0)),
                pl.BlockSpec((1, FF, H), lambda i, be: (be[i], 0, 0)),
                pl.BlockSpec((1, H, FF), lambda i, be: (be[i], 0, 0)),
                pl.BlockSpec((BLK, WV), lambda i, be: (i, 0)),
            ],
            out_specs=pl.BlockSpec((BLK, H), lambda i, be: (i, 0)),
        ),
        out_shape=jax.ShapeDtypeStruct((NPAD, H), _f32),
    )(be, xs, gate_w, up_w, down_w, ws)


_TPW = T // 32
_CT = 16
_NCT = _TPW // _CT

def _combine_body(y_hbm, d_hbm, out_hbm,
                  i0a, i0b, i1a, i1b, r0a, r0b, r1a, r1b,
                  semi0, semi1, semo0, semo1):
    wid = lax.axis_index("s") * 2 + lax.axis_index("c")
    tb = wid * _TPW
    idx0 = (i0a, i0b)
    idx1 = (i1a, i1b)
    r0 = (r0a, r0b)
    r1 = (r1a, r1b)
    semi = (semi0, semi1)
    semo = (semo0, semo1)

    def start_in(c, b):
        t0 = tb + c * _CT
        cpa = pltpu.async_copy(d_hbm.at[pl.ds(t0, _CT)], idx0[b], semi[b])
        cpb = pltpu.async_copy(d_hbm.at[pl.ds(T + t0, _CT)], idx1[b], semi[b])
        cpa.wait()
        cpb.wait()
        return (pltpu.async_copy(y_hbm.at[idx0[b]], r0[b], semi[b]),
                pltpu.async_copy(y_hbm.at[idx1[b]], r1[b], semi[b]))

    pend_in = {0: start_in(0, 0)}
    pend_out = {}
    for c in range(_NCT):
        b = c & 1
        for cp in pend_in.pop(c):
            cp.wait()
        if c + 1 < _NCT:
            if c - 1 >= 0:
                pend_out.pop(c - 1).wait()
            pend_in[c + 1] = start_in(c + 1, (c + 1) & 1)

        def row_body(r, carry):
            for k in range(H // 16):
                sl = pl.ds(k * 16, 16)
                r0[b][r, sl] = r0[b][r, sl] + r1[b][r, sl]
            return carry

        lax.fori_loop(0, _CT, row_body, 0)
        pend_out[c] = pltpu.async_copy(
            r0[b], out_hbm.at[pl.ds(tb + c * _CT, _CT)], semo[b])
    for k in sorted(pend_out):
        pend_out[k].wait()


def _combine(y, d_pairs):
    mesh = plsc.VectorSubcoreMesh(core_axis_name="c", subcore_axis_name="s")
    return pl.kernel(
        _combine_body,
        out_type=jax.ShapeDtypeStruct((T, H), _f32),
        mesh=mesh,
        scratch_types=[pltpu.VMEM((_CT,), _i32),
                       pltpu.VMEM((_CT,), _i32),
                       pltpu.VMEM((_CT,), _i32),
                       pltpu.VMEM((_CT,), _i32),
                       pltpu.VMEM((_CT, H), _f32),
                       pltpu.VMEM((_CT, H), _f32),
                       pltpu.VMEM((_CT, H), _f32),
                       pltpu.VMEM((_CT, H), _f32),
                       pltpu.SemaphoreType.DMA,
                       pltpu.SemaphoreType.DMA,
                       pltpu.SemaphoreType.DMA,
                       pltpu.SemaphoreType.DMA],
    )(y, d_pairs)


def kernel(hidden_states, router_w, gate_w, up_w, down_w):
    b, s, h = hidden_states.shape
    x = hidden_states.reshape(-1, h)
    d, w, be, lb, z = _router(x, router_w)
    d_pairs = jnp.concatenate([d[0], d[1]])
    w_pairs = jnp.broadcast_to(
        jnp.concatenate([w[0], w[1]])[:, None], (KTOP * T, WV))
    be_vec = be[0, :NB]
    xs, ws = _dispatch(x, w_pairs, d_pairs)
    y = _ffn(be_vec, xs, ws, _cast_bf16(gate_w, 1), _cast_bf16(up_w, 1),
             _cast_bf16(down_w, 2))
    final = _combine(y, d_pairs)
    return (final.reshape(b, s, h), lb.reshape(()), z.reshape(()))


# speedup vs baseline: 1.2470x; 1.0475x over previous
import functools

import jax
import jax.numpy as jnp
from jax import lax
from jax.experimental import pallas as pl
from jax.experimental.pallas import tpu as pltpu
from jax.experimental.pallas import tpu_sc as plsc

T = 4096
H = 1024
E = 8
FF = 2816
KTOP = 2
AUX_COEF = 0.01
Z_COEF = 0.001

BLK = 256
NB = (KTOP * T) // BLK + E - 1
NPAD = NB * BLK
NFF = 2
FFC = FF // NFF
TB = 512
NTB = T // TB
WV = 128

_f32 = jnp.float32
_i32 = jnp.int32


def _router_body(x_ref, rw_ref, d_ref, w_ref, be_ref, lb_ref, z_ref,
                 oh1_ref, oh2_ref):
    x = x_ref[...]
    rw = rw_ref[...]
    logits = lax.dot_general(rw, x, (((1,), (1,)), ((), ())),
                             preferred_element_type=_f32)
    ii = lax.broadcasted_iota(_i32, (E, T), 0)
    m1 = jnp.max(logits, axis=0, keepdims=True)
    i1 = jnp.min(jnp.where(logits == m1, ii, E), axis=0, keepdims=True)
    oh1 = (ii == i1).astype(_f32)
    masked = jnp.where(oh1 > 0.5, -jnp.inf, logits)
    m2 = jnp.max(masked, axis=0, keepdims=True)
    i2 = jnp.min(jnp.where(masked == m2, ii, E), axis=0, keepdims=True)
    oh2 = (ii == i2).astype(_f32)
    oh1_ref[...] = oh1
    oh2_ref[...] = oh2

    ew = jnp.exp(m2 - m1)
    w_ref[0:1, :] = 1.0 / (1.0 + ew)
    w_ref[1:2, :] = ew / (1.0 + ew)

    se = jnp.sum(jnp.exp(logits - m1), axis=0, keepdims=True)
    lse = m1 + jnp.log(se)
    z_ref[...] = jnp.reshape(Z_COEF * jnp.sum(lse * lse) / T, (1, 1))

    counts = jnp.sum(oh1 + oh2, axis=1, keepdims=True)
    usage = counts / T
    mean_u = jnp.sum(usage) / E
    var_u = jnp.sum((usage - mean_u) ** 2) / (E - 1)
    lb_ref[...] = jnp.reshape(AUX_COEF * E * var_u, (1, 1))

    nb_e = jnp.floor((counts + (BLK - 1)) / BLK)
    er = lax.broadcasted_iota(_i32, (E, E), 0)
    ec = lax.broadcasted_iota(_i32, (E, E), 1)
    mlt = (ec < er).astype(_f32)
    cumx = lax.dot_general(mlt, nb_e, (((1,), (0,)), ((), ())),
                           preferred_element_type=_f32,
                           precision=lax.Precision.HIGHEST)
    pad_off = BLK * cumx
    c0tot = jnp.sum(oh1, axis=1, keepdims=True)

    cum_incl = jnp.round(cumx + nb_e).astype(_i32)
    bidx = lax.broadcasted_iota(_i32, (E, 64), 1)
    be = jnp.sum((bidx >= cum_incl).astype(_i32), axis=0, keepdims=True)
    be_ref[0:1, :] = jnp.minimum(be, E - 1)

    tr = lax.broadcasted_iota(_i32, (TB, TB), 0)
    tc = lax.broadcasted_iota(_i32, (TB, TB), 1)
    triu = (tr < tc).astype(_f32)

    def body(b, carry):
        run0, run1 = carry
        sl = pl.ds(b * TB, TB)
        o1 = oh1_ref[:, sl]
        o2 = oh2_ref[:, sl]
        r0 = lax.dot_general(o1, triu, (((1,), (0,)), ((), ())),
                             preferred_element_type=_f32,
                             precision=lax.Precision.HIGHEST)
        r1 = lax.dot_general(o2, triu, (((1,), (0,)), ((), ())),
                             preferred_element_type=_f32,
                             precision=lax.Precision.HIGHEST)
        d0 = jnp.sum(o1 * (pad_off + run0) + o1 * r0, axis=0,
                     keepdims=True)
        d1 = jnp.sum(o2 * (pad_off + c0tot + run1) + o2 * r1, axis=0,
                     keepdims=True)
        d_ref[0:1, sl] = jnp.round(d0).astype(_i32)
        d_ref[1:2, sl] = jnp.round(d1).astype(_i32)
        return (run0 + jnp.sum(o1, axis=1, keepdims=True),
                run1 + jnp.sum(o2, axis=1, keepdims=True))

    lax.fori_loop(0, NTB, body, (jnp.zeros((E, 1), _f32),
                                 jnp.zeros((E, 1), _f32)))


def _router(x, router_w):
    return pl.pallas_call(
        _router_body,
        out_shape=(
            jax.ShapeDtypeStruct((8, T), _i32),
            jax.ShapeDtypeStruct((8, T), _f32),
            jax.ShapeDtypeStruct((8, 64), _i32),
            jax.ShapeDtypeStruct((1, 1), _f32),
            jax.ShapeDtypeStruct((1, 1), _f32),
        ),
        scratch_shapes=[pltpu.VMEM((E, T), _f32), pltpu.VMEM((E, T), _f32)],
    )(x, router_w)


_PPW = (KTOP * T) // 32
_CH = 32
_NCH = _PPW // _CH

def _dispatch_body(x_hbm, w_hbm, d_hbm, xs_hbm, ws_hbm,
                   idx0, idx1, rows0, rows1, w0, w1,
                   semi0, semi1, semo0, semo1):
    wid = lax.axis_index("s") * 2 + lax.axis_index("c")
    base = wid * _PPW
    idx = (idx0, idx1)
    rows = (rows0, rows1)
    wv = (w0, w1)
    semi = (semi0, semi1)
    semo = (semo0, semo1)

    def start_in(c, b):
        off = base + c * _CH
        src = lax.rem(off, T)
        return (pltpu.async_copy(d_hbm.at[pl.ds(off, _CH)], idx[b], semi[b]),
                pltpu.async_copy(x_hbm.at[pl.ds(src, _CH)], rows[b], semi[b]),
                pltpu.async_copy(w_hbm.at[pl.ds(off, _CH)], wv[b], semi[b]))

    pend_in = {0: start_in(0, 0)}
    pend_out = {}
    for c in range(_NCH):
        b = c & 1
        for cp in pend_in.pop(c):
            cp.wait()
        if c + 1 < _NCH:
            if c - 1 >= 0:
                for cp in pend_out.pop(c - 1):
                    cp.wait()
            pend_in[c + 1] = start_in(c + 1, (c + 1) & 1)
        pend_out[c] = (
            pltpu.async_copy(rows[b], xs_hbm.at[idx[b]], semo[b]),
            pltpu.async_copy(wv[b], ws_hbm.at[idx[b]], semo[b]))
    for k in sorted(pend_out):
        for cp in pend_out[k]:
            cp.wait()


def _dispatch(x, w_pairs, d_pairs):
    mesh = plsc.VectorSubcoreMesh(core_axis_name="c", subcore_axis_name="s")
    return pl.kernel(
        _dispatch_body,
        out_type=(jax.ShapeDtypeStruct((NPAD, H), _f32),
                  jax.ShapeDtypeStruct((NPAD, WV), _f32)),
        mesh=mesh,
        scratch_types=[pltpu.VMEM((_CH,), _i32),
                       pltpu.VMEM((_CH,), _i32),
                       pltpu.VMEM((_CH, H), _f32),
                       pltpu.VMEM((_CH, H), _f32),
                       pltpu.VMEM((_CH, WV), _f32),
                       pltpu.VMEM((_CH, WV), _f32),
                       pltpu.SemaphoreType.DMA,
                       pltpu.SemaphoreType.DMA,
                       pltpu.SemaphoreType.DMA,
                       pltpu.SemaphoreType.DMA],
    )(x, w_pairs, d_pairs)


def _cast_mid_body(i_ref, o_ref):
    o_ref[...] = i_ref[...].astype(jnp.bfloat16)


def _cast_bf16(arr, mid_axis):
    e, d1, d2 = arr.shape
    nc = NFF
    if mid_axis == 1:
        blk, imap = (1, d1 // nc, d2), (lambda i, j: (i, j, 0))
    else:
        blk, imap = (1, d1, d2 // nc), (lambda i, j: (i, 0, j))
    return pl.pallas_call(
        _cast_mid_body,
        grid=(e, nc),
        in_specs=[pl.BlockSpec(blk, imap)],
        out_specs=pl.BlockSpec(blk, imap),
        out_shape=jax.ShapeDtypeStruct(arr.shape, jnp.bfloat16),
    )(arr)


def _ffn_body(be_ref, x_ref, gw_ref, uw_ref, dw_ref, w_ref, y_ref):
    x = x_ref[...].astype(jnp.bfloat16)
    g = lax.dot_general(x, gw_ref[0], (((1,), (1,)), ((), ())),
                        preferred_element_type=_f32)
    u = lax.dot_general(x, uw_ref[0], (((1,), (1,)), ((), ())),
                        preferred_element_type=_f32)
    act = (g * (1.0 / (1.0 + jnp.exp(-g))) * u).astype(jnp.bfloat16)
    y = lax.dot_general(act, dw_ref[0], (((1,), (1,)), ((), ())),
                        preferred_element_type=_f32)
    y_ref[...] = y * w_ref[:, 0:1]


def _ffn(be, xs, ws, gate_w, up_w, down_w):
    return pl.pallas_call(
        _ffn_body,
        grid_spec=pltpu.PrefetchScalarGridSpec(
            num_scalar_prefetch=1,
            grid=(NB,),
            in_specs=[
                pl.BlockSpec((BLK, H), lambda i, be: (i, 0)),
                pl.BlockSpec((1, FF, H), lambda i, be: (be[i], 0, 0)),
                pl.BlockSpec((1, FF, H), lambda i, be: (be[i], 0, 0)),
                pl.BlockSpec((1, H, FF), lambda i, be: (be[i], 0, 0)),
                pl.BlockSpec((BLK, WV), lambda i, be: (i, 0)),
            ],
            out_specs=pl.BlockSpec((BLK, H), lambda i, be: (i, 0)),
        ),
        out_shape=jax.ShapeDtypeStruct((NPAD, H), _f32),
    )(be, xs, gate_w, up_w, down_w, ws)


_TPW = T // 32
_CT = 16
_NCT = _TPW // _CT

def _combine_body(y_hbm, d_hbm, out_hbm,
                  i0a, i0b, i1a, i1b, r0a, r0b, r1a, r1b,
                  semi0, semi1, semo0, semo1):
    wid = lax.axis_index("s") * 2 + lax.axis_index("c")
    tb = wid * _TPW
    idx0 = (i0a, i0b)
    idx1 = (i1a, i1b)
    r0 = (r0a, r0b)
    r1 = (r1a, r1b)
    semi = (semi0, semi1)
    semo = (semo0, semo1)

    def start_in(c, b):
        t0 = tb + c * _CT
        cpa = pltpu.async_copy(d_hbm.at[pl.ds(t0, _CT)], idx0[b], semi[b])
        cpb = pltpu.async_copy(d_hbm.at[pl.ds(T + t0, _CT)], idx1[b], semi[b])
        cpa.wait()
        cpb.wait()
        return (pltpu.async_copy(y_hbm.at[idx0[b]], r0[b], semi[b]),
                pltpu.async_copy(y_hbm.at[idx1[b]], r1[b], semi[b]))

    pend_in = {0: start_in(0, 0)}
    pend_out = {}
    for c in range(_NCT):
        b = c & 1
        for cp in pend_in.pop(c):
            cp.wait()
        if c + 1 < _NCT:
            if c - 1 >= 0:
                pend_out.pop(c - 1).wait()
            pend_in[c + 1] = start_in(c + 1, (c + 1) & 1)

        def row_body(r, carry):
            for k in range(H // 16):
                sl = pl.ds(k * 16, 16)
                r0[b][r, sl] = r0[b][r, sl] + r1[b][r, sl]
            return carry

        lax.fori_loop(0, _CT, row_body, 0)
        pend_out[c] = pltpu.async_copy(
            r0[b], out_hbm.at[pl.ds(tb + c * _CT, _CT)], semo[b])
    for k in sorted(pend_out):
        pend_out[k].wait()


def _combine(y, d_pairs):
    mesh = plsc.VectorSubcoreMesh(core_axis_name="c", subcore_axis_name="s")
    return pl.kernel(
        _combine_body,
        out_type=jax.ShapeDtypeStruct((T, H), _f32),
        mesh=mesh,
        scratch_types=[pltpu.VMEM((_CT,), _i32),
                       pltpu.VMEM((_CT,), _i32),
                       pltpu.VMEM((_CT,), _i32),
                       pltpu.VMEM((_CT,), _i32),
                       pltpu.VMEM((_CT, H), _f32),
                       pltpu.VMEM((_CT, H), _f32),
                       pltpu.VMEM((_CT, H), _f32),
                       pltpu.VMEM((_CT, H), _f32),
                       pltpu.SemaphoreType.DMA,
                       pltpu.SemaphoreType.DMA,
                       pltpu.SemaphoreType.DMA,
                       pltpu.SemaphoreType.DMA],
    )(y, d_pairs)


def kernel(hidden_states, router_w, gate_w, up_w, down_w):
    b, s, h = hidden_states.shape
    x = hidden_states.reshape(-1, h)
    d, w, be, lb, z = _router(x, router_w)
    d_pairs = jnp.concatenate([d[0], d[1]])
    w_pairs = jnp.broadcast_to(
        jnp.concatenate([w[0], w[1]])[:, None], (KTOP * T, WV))
    be_vec = be[0, :NB]
    xs, ws = _dispatch(x, w_pairs, d_pairs)
    y = _ffn(be_vec, xs, ws, _cast_bf16(gate_w, 1), _cast_bf16(up_w, 1),
             _cast_bf16(down_w, 2))
    final = _combine(y, d_pairs)
    return (final.reshape(b, s, h), lb.reshape(()), z.reshape(()))


# speedup vs baseline: 1.2499x; 1.0023x over previous
import functools

import jax
import jax.numpy as jnp
from jax import lax
from jax.experimental import pallas as pl
from jax.experimental.pallas import tpu as pltpu
from jax.experimental.pallas import tpu_sc as plsc

T = 4096
H = 1024
E = 8
FF = 2816
KTOP = 2
AUX_COEF = 0.01
Z_COEF = 0.001

BLK = 256
NB = (KTOP * T) // BLK + E - 1
NPAD = NB * BLK
NFF = 2
FFC = FF // NFF
TB = 512
NTB = T // TB
WV = 128

_f32 = jnp.float32
_i32 = jnp.int32


def _router_body(x_ref, rw_ref, d_ref, w_ref, be_ref, lb_ref, z_ref,
                 oh1_ref, oh2_ref):
    x = x_ref[...]
    rw = rw_ref[...]
    logits = lax.dot_general(rw, x, (((1,), (1,)), ((), ())),
                             preferred_element_type=_f32)
    ii = lax.broadcasted_iota(_i32, (E, T), 0)
    m1 = jnp.max(logits, axis=0, keepdims=True)
    i1 = jnp.min(jnp.where(logits == m1, ii, E), axis=0, keepdims=True)
    oh1 = (ii == i1).astype(_f32)
    masked = jnp.where(oh1 > 0.5, -jnp.inf, logits)
    m2 = jnp.max(masked, axis=0, keepdims=True)
    i2 = jnp.min(jnp.where(masked == m2, ii, E), axis=0, keepdims=True)
    oh2 = (ii == i2).astype(_f32)
    oh1_ref[...] = oh1
    oh2_ref[...] = oh2

    ew = jnp.exp(m2 - m1)
    w_ref[0:1, :] = 1.0 / (1.0 + ew)
    w_ref[1:2, :] = ew / (1.0 + ew)

    se = jnp.sum(jnp.exp(logits - m1), axis=0, keepdims=True)
    lse = m1 + jnp.log(se)
    z_ref[...] = jnp.reshape(Z_COEF * jnp.sum(lse * lse) / T, (1, 1))

    counts = jnp.sum(oh1 + oh2, axis=1, keepdims=True)
    usage = counts / T
    mean_u = jnp.sum(usage) / E
    var_u = jnp.sum((usage - mean_u) ** 2) / (E - 1)
    lb_ref[...] = jnp.reshape(AUX_COEF * E * var_u, (1, 1))

    nb_e = jnp.floor((counts + (BLK - 1)) / BLK)
    er = lax.broadcasted_iota(_i32, (E, E), 0)
    ec = lax.broadcasted_iota(_i32, (E, E), 1)
    mlt = (ec < er).astype(_f32)
    cumx = lax.dot_general(mlt, nb_e, (((1,), (0,)), ((), ())),
                           preferred_element_type=_f32,
                           precision=lax.Precision.HIGHEST)
    pad_off = BLK * cumx
    c0tot = jnp.sum(oh1, axis=1, keepdims=True)

    cum_incl = jnp.round(cumx + nb_e).astype(_i32)
    bidx = lax.broadcasted_iota(_i32, (E, 64), 1)
    be = jnp.sum((bidx >= cum_incl).astype(_i32), axis=0, keepdims=True)
    be_ref[0:1, :] = jnp.minimum(be, E - 1)

    tr = lax.broadcasted_iota(_i32, (TB, TB), 0)
    tc = lax.broadcasted_iota(_i32, (TB, TB), 1)
    triu = (tr < tc).astype(_f32)

    def body(b, carry):
        run0, run1 = carry
        sl = pl.ds(b * TB, TB)
        o1 = oh1_ref[:, sl]
        o2 = oh2_ref[:, sl]
        r0 = lax.dot_general(o1, triu, (((1,), (0,)), ((), ())),
                             preferred_element_type=_f32,
                             precision=lax.Precision.HIGHEST)
        r1 = lax.dot_general(o2, triu, (((1,), (0,)), ((), ())),
                             preferred_element_type=_f32,
                             precision=lax.Precision.HIGHEST)
        d0 = jnp.sum(o1 * (pad_off + run0) + o1 * r0, axis=0,
                     keepdims=True)
        d1 = jnp.sum(o2 * (pad_off + c0tot + run1) + o2 * r1, axis=0,
                     keepdims=True)
        d_ref[0:1, sl] = jnp.round(d0).astype(_i32)
        d_ref[1:2, sl] = jnp.round(d1).astype(_i32)
        return (run0 + jnp.sum(o1, axis=1, keepdims=True),
                run1 + jnp.sum(o2, axis=1, keepdims=True))

    lax.fori_loop(0, NTB, body, (jnp.zeros((E, 1), _f32),
                                 jnp.zeros((E, 1), _f32)))


def _router(x, router_w):
    return pl.pallas_call(
        _router_body,
        out_shape=(
            jax.ShapeDtypeStruct((8, T), _i32),
            jax.ShapeDtypeStruct((8, T), _f32),
            jax.ShapeDtypeStruct((8, 64), _i32),
            jax.ShapeDtypeStruct((1, 1), _f32),
            jax.ShapeDtypeStruct((1, 1), _f32),
        ),
        scratch_shapes=[pltpu.VMEM((E, T), _f32), pltpu.VMEM((E, T), _f32)],
    )(x, router_w)


_PPW = (KTOP * T) // 32
_CH = 32
_NCH = _PPW // _CH

def _dispatch_body(x_hbm, w_hbm, d_hbm, xs_hbm, ws_hbm,
                   idx0, idx1, rows0, rows1, w0, w1,
                   semi0, semi1, semo0, semo1):
    wid = lax.axis_index("s") * 2 + lax.axis_index("c")
    base = wid * _PPW
    idx = (idx0, idx1)
    rows = (rows0, rows1)
    wv = (w0, w1)
    semi = (semi0, semi1)
    semo = (semo0, semo1)

    def start_in(c, b):
        off = base + c * _CH
        src = lax.rem(off, T)
        return (pltpu.async_copy(d_hbm.at[pl.ds(off, _CH)], idx[b], semi[b]),
                pltpu.async_copy(x_hbm.at[pl.ds(src, _CH)], rows[b], semi[b]),
                pltpu.async_copy(w_hbm.at[pl.ds(off, _CH)], wv[b], semi[b]))

    pend_in = {0: start_in(0, 0)}
    pend_out = {}
    for c in range(_NCH):
        b = c & 1
        for cp in pend_in.pop(c):
            cp.wait()
        if c + 1 < _NCH:
            if c - 1 >= 0:
                for cp in pend_out.pop(c - 1):
                    cp.wait()
            pend_in[c + 1] = start_in(c + 1, (c + 1) & 1)
        pend_out[c] = (
            pltpu.async_copy(rows[b], xs_hbm.at[idx[b]], semo[b]),
            pltpu.async_copy(wv[b], ws_hbm.at[idx[b]], semo[b]))
    for k in sorted(pend_out):
        for cp in pend_out[k]:
            cp.wait()


def _dispatch(x, w_pairs, d_pairs):
    mesh = plsc.VectorSubcoreMesh(core_axis_name="c", subcore_axis_name="s")
    return pl.kernel(
        _dispatch_body,
        out_type=(jax.ShapeDtypeStruct((NPAD, H), _f32),
                  jax.ShapeDtypeStruct((NPAD, WV), _f32)),
        mesh=mesh,
        scratch_types=[pltpu.VMEM((_CH,), _i32),
                       pltpu.VMEM((_CH,), _i32),
                       pltpu.VMEM((_CH, H), _f32),
                       pltpu.VMEM((_CH, H), _f32),
                       pltpu.VMEM((_CH, WV), _f32),
                       pltpu.VMEM((_CH, WV), _f32),
                       pltpu.SemaphoreType.DMA,
                       pltpu.SemaphoreType.DMA,
                       pltpu.SemaphoreType.DMA,
                       pltpu.SemaphoreType.DMA],
    )(x, w_pairs, d_pairs)


def _cast_body(g_ref, u_ref, d_ref, go_ref, uo_ref, do_ref):
    bf = jnp.bfloat16
    go_ref[...] = g_ref[...].astype(bf)
    uo_ref[...] = u_ref[...].astype(bf)
    do_ref[...] = d_ref[...].astype(bf)


def _cast_weights(gate_w, up_w, down_w):
    bf = jnp.bfloat16
    mid = pl.BlockSpec((1, FFC, H), lambda i, j: (i, j, 0))
    last = pl.BlockSpec((1, H, FFC), lambda i, j: (i, 0, j))
    return pl.pallas_call(
        _cast_body,
        grid=(E, NFF),
        in_specs=[mid, mid, last],
        out_specs=[mid, mid, last],
        out_shape=(jax.ShapeDtypeStruct((E, FF, H), bf),
                   jax.ShapeDtypeStruct((E, FF, H), bf),
                   jax.ShapeDtypeStruct((E, H, FF), bf)),
    )(gate_w, up_w, down_w)


def _ffn_body(be_ref, x_ref, gw_ref, uw_ref, dw_ref, w_ref, y_ref):
    x = x_ref[...].astype(jnp.bfloat16)
    g = lax.dot_general(x, gw_ref[0], (((1,), (1,)), ((), ())),
                        preferred_element_type=_f32)
    u = lax.dot_general(x, uw_ref[0], (((1,), (1,)), ((), ())),
                        preferred_element_type=_f32)
    act = (g * (1.0 / (1.0 + jnp.exp(-g))) * u).astype(jnp.bfloat16)
    y = lax.dot_general(act, dw_ref[0], (((1,), (1,)), ((), ())),
                        preferred_element_type=_f32)
    y_ref[...] = y * w_ref[:, 0:1]


def _ffn(be, xs, ws, gate_w, up_w, down_w):
    return pl.pallas_call(
        _ffn_body,
        grid_spec=pltpu.PrefetchScalarGridSpec(
            num_scalar_prefetch=1,
            grid=(NB,),
            in_specs=[
                pl.BlockSpec((BLK, H), lambda i, be: (i, 0)),
                pl.BlockSpec((1, FF, H), lambda i, be: (be[i], 0, 0)),
                pl.BlockSpec((1, FF, H), lambda i, be: (be[i], 0, 0)),
                pl.BlockSpec((1, H, FF), lambda i, be: (be[i], 0, 0)),
                pl.BlockSpec((BLK, WV), lambda i, be: (i, 0)),
            ],
            out_specs=pl.BlockSpec((BLK, H), lambda i, be: (i, 0)),
        ),
        out_shape=jax.ShapeDtypeStruct((NPAD, H), _f32),
    )(be, xs, gate_w, up_w, down_w, ws)


_TPW = T // 32
_CT = 16
_NCT = _TPW // _CT

def _combine_body(y_hbm, d_hbm, out_hbm,
                  i0a, i0b, i1a, i1b, r0a, r0b, r1a, r1b,
                  semi0, semi1, semo0, semo1):
    wid = lax.axis_index("s") * 2 + lax.axis_index("c")
    tb = wid * _TPW
    idx0 = (i0a, i0b)
    idx1 = (i1a, i1b)
    r0 = (r0a, r0b)
    r1 = (r1a, r1b)
    semi = (semi0, semi1)
    semo = (semo0, semo1)

    def start_in(c, b):
        t0 = tb + c * _CT
        cpa = pltpu.async_copy(d_hbm.at[pl.ds(t0, _CT)], idx0[b], semi[b])
        cpb = pltpu.async_copy(d_hbm.at[pl.ds(T + t0, _CT)], idx1[b], semi[b])
        cpa.wait()
        cpb.wait()
        return (pltpu.async_copy(y_hbm.at[idx0[b]], r0[b], semi[b]),
                pltpu.async_copy(y_hbm.at[idx1[b]], r1[b], semi[b]))

    pend_in = {0: start_in(0, 0)}
    pend_out = {}
    for c in range(_NCT):
        b = c & 1
        for cp in pend_in.pop(c):
            cp.wait()
        if c + 1 < _NCT:
            if c - 1 >= 0:
                pend_out.pop(c - 1).wait()
            pend_in[c + 1] = start_in(c + 1, (c + 1) & 1)

        def row_body(r, carry):
            for k in range(H // 16):
                sl = pl.ds(k * 16, 16)
                r0[b][r, sl] = r0[b][r, sl] + r1[b][r, sl]
            return carry

        lax.fori_loop(0, _CT, row_body, 0)
        pend_out[c] = pltpu.async_copy(
            r0[b], out_hbm.at[pl.ds(tb + c * _CT, _CT)], semo[b])
    for k in sorted(pend_out):
        pend_out[k].wait()


def _combine(y, d_pairs):
    mesh = plsc.VectorSubcoreMesh(core_axis_name="c", subcore_axis_name="s")
    return pl.kernel(
        _combine_body,
        out_type=jax.ShapeDtypeStruct((T, H), _f32),
        mesh=mesh,
        scratch_types=[pltpu.VMEM((_CT,), _i32),
                       pltpu.VMEM((_CT,), _i32),
                       pltpu.VMEM((_CT,), _i32),
                       pltpu.VMEM((_CT,), _i32),
                       pltpu.VMEM((_CT, H), _f32),
                       pltpu.VMEM((_CT, H), _f32),
                       pltpu.VMEM((_CT, H), _f32),
                       pltpu.VMEM((_CT, H), _f32),
                       pltpu.SemaphoreType.DMA,
                       pltpu.SemaphoreType.DMA,
                       pltpu.SemaphoreType.DMA,
                       pltpu.SemaphoreType.DMA],
    )(y, d_pairs)


def kernel(hidden_states, router_w, gate_w, up_w, down_w):
    b, s, h = hidden_states.shape
    x = hidden_states.reshape(-1, h)
    d, w, be, lb, z = _router(x, router_w)
    d_pairs = jnp.concatenate([d[0], d[1]])
    w_pairs = jnp.broadcast_to(
        jnp.concatenate([w[0], w[1]])[:, None], (KTOP * T, WV))
    be_vec = be[0, :NB]
    xs, ws = _dispatch(x, w_pairs, d_pairs)
    gw_bf, uw_bf, dw_bf = _cast_weights(gate_w, up_w, down_w)
    y = _ffn(be_vec, xs, ws, gw_bf, uw_bf, dw_bf)
    final = _combine(y, d_pairs)
    return (final.reshape(b, s, h), lb.reshape(()), z.reshape(()))
